# baseline (speedup 1.0000x reference)
import jax
import jax.numpy as jnp
from jax import lax
from jax.experimental import pallas as pl
from jax.experimental.pallas import tpu as pltpu
from jax.experimental.pallas import tpu_sc as plsc

N_NODES = 10000
N_EDGES = 320000
D_MSG = 16

NP = 10240
GSZ = 128
GROUPS = N_EDGES // GSZ
NC, NS = 2, 16
NW = NC * NS
GPW = 78
CH = 6
NCHUNK = GPW // CH
RPW = NP // NS
PKW = RPW // 8

_mesh = plsc.VectorSubcoreMesh(core_axis_name="c", subcore_axis_name="s")
_sc_params = pltpu.CompilerParams(use_tc_tiling_on_sc=False,
                                  needs_layout_passes=False)


def _newton_rsqrt(d):
    iv = plsc.bitcast(d, jnp.int32)
    y = plsc.bitcast(jnp.int32(0x5F3759DF) - (iv >> 1), jnp.float32)
    for _ in range(3):
        y = y * (1.5 - 0.5 * d * y * y)
    return y


def _worker_base(wid):
    return wid * GPW + jnp.minimum(wid, 4)


def _load_extra_idx(adj_hbm, srcv, dstv, wid, base):
    @pl.when(wid < 4)
    def _():
        pltpu.sync_copy(adj_hbm.at[0].at[pl.ds(base + GPW, 1)],
                        srcv.at[pl.ds(GPW, 1)])
        pltpu.sync_copy(adj_hbm.at[1].at[pl.ds(base + GPW, 1)],
                        dstv.at[pl.ds(GPW, 1)])


def _msg_pipeline(srcv, dstv, rowsbuf, table_sh, acc_sh, gsem, ssem, wid):
    @pl.when(wid < 4)
    def _():
        pltpu.async_copy(table_sh.at[srcv.at[GPW]],
                         rowsbuf.at[0].at[0], gsem).wait()
        pltpu.async_copy(rowsbuf.at[0].at[0], acc_sh.at[dstv.at[GPW]],
                         ssem, add=True).wait()

    def fire_gathers(c, b):
        return [pltpu.async_copy(table_sh.at[srcv.at[c * CH + j]],
                                 rowsbuf.at[b].at[j], gsem)
                for j in range(CH)]

    def fire_scatters(c, b):
        return [pltpu.async_copy(rowsbuf.at[b].at[j],
                                 acc_sh.at[dstv.at[c * CH + j]], ssem,
                                 add=True)
                for j in range(CH)]

    g = fire_gathers(0, 0)
    s_prev = None
    for c in range(NCHUNK):
        b = c % 2
        for d_ in g:
            d_.wait()
        s = fire_scatters(c, b)
        if s_prev is not None:
            for d_ in s_prev:
                d_.wait()
        if c + 1 < NCHUNK:
            g = fire_gathers(c + 1, 1 - b)
        s_prev = s
    for d_ in s_prev:
        d_.wait()


def _deg_body(adj_hbm, ones_hbm, zeros_hbm, out_hbm, dstv, srcv, onesb,
              acc_sh, ssem):
    cid = lax.axis_index("c")
    sid = lax.axis_index("s")
    wid = sid * NC + cid
    base = _worker_base(wid)
    rs = pl.ds(sid * RPW, RPW)

    pltpu.sync_copy(zeros_hbm.at[rs], acc_sh.at[rs])
    pltpu.sync_copy(ones_hbm, onesb)
    pltpu.sync_copy(adj_hbm.at[1].at[pl.ds(base, GPW)],
                    dstv.at[pl.ds(0, GPW)])
    _load_extra_idx(adj_hbm, srcv, dstv, wid, base)
    plsc.subcore_barrier()

    @pl.when(wid < 4)
    def _():
        pltpu.async_copy(onesb, acc_sh.at[dstv.at[GPW]], ssem,
                         add=True).wait()

    s_prev = None
    for c in range(NCHUNK):
        s = [pltpu.async_copy(onesb, acc_sh.at[dstv.at[c * CH + j]], ssem,
                              add=True)
             for j in range(CH)]
        if s_prev is not None:
            for d_ in s_prev:
                d_.wait()
        s_prev = s
    for d_ in s_prev:
        d_.wait()

    plsc.subcore_barrier()
    pltpu.sync_copy(acc_sh.at[rs], out_hbm.at[cid].at[rs])


def _msg1_body(degp_hbm, h1_hbm, adj_hbm, out_hbm,
               srcv, dstv, rowsbuf, va, vb, vh, acc_sh, table_sh,
               gsem, ssem):
    cid = lax.axis_index("c")
    sid = lax.axis_index("s")
    wid = sid * NC + cid
    base = _worker_base(wid)
    rs = pl.ds(sid * RPW, RPW)

    pltpu.sync_copy(degp_hbm.at[0].at[rs], va)
    pltpu.sync_copy(degp_hbm.at[1].at[rs], vb)
    pltpu.sync_copy(h1_hbm.at[pl.ds(sid * PKW, PKW)], vh)

    @pl.loop(0, PKW)
    def _(i):
        for u in range(8):
            r = i * 8 + u
            y = _newton_rsqrt(va[r] + vb[r] + 1.0)
            vb[r] = vh[i, pl.ds(u * D_MSG, D_MSG)] * y
            va[r] = jnp.zeros((D_MSG,), jnp.float32)

    pltpu.sync_copy(vb, table_sh.at[rs])
    pltpu.sync_copy(va, acc_sh.at[rs])
    pltpu.sync_copy(adj_hbm.at[0].at[pl.ds(base, GPW)],
                    srcv.at[pl.ds(0, GPW)])
    pltpu.sync_copy(adj_hbm.at[1].at[pl.ds(base, GPW)],
                    dstv.at[pl.ds(0, GPW)])
    _load_extra_idx(adj_hbm, srcv, dstv, wid, base)
    plsc.subcore_barrier()

    _msg_pipeline(srcv, dstv, rowsbuf, table_sh, acc_sh, gsem, ssem, wid)

    plsc.subcore_barrier()
    pltpu.sync_copy(acc_sh.at[rs], out_hbm.at[cid].at[rs])


def _msg2_body(degp_hbm, h1_hbm, acc1_hbm, adj_hbm, out_hbm,
               srcv, dstv, rowsbuf, va, vb, vc, vd, vh, acc_sh, table_sh,
               gsem, ssem):
    cid = lax.axis_index("c")
    sid = lax.axis_index("s")
    wid = sid * NC + cid
    base = _worker_base(wid)
    rs = pl.ds(sid * RPW, RPW)

    pltpu.sync_copy(degp_hbm.at[0].at[rs], va)
    pltpu.sync_copy(degp_hbm.at[1].at[rs], vb)
    pltpu.sync_copy(acc1_hbm.at[0].at[rs], vc)
    pltpu.sync_copy(acc1_hbm.at[1].at[rs], vd)
    pltpu.sync_copy(h1_hbm.at[pl.ds(sid * PKW, PKW)], vh)

    @pl.loop(0, PKW)
    def _(i):
        for u in range(8):
            r = i * 8 + u
            y = _newton_rsqrt(va[r] + vb[r] + 1.0)
            t = (vc[r] + vd[r]) * y + vh[i, pl.ds(u * D_MSG, D_MSG)] * (y * y)
            va[r] = y
            vb[r] = jnp.maximum(t, 0.0) * y
            vc[r] = jnp.zeros((D_MSG,), jnp.float32)

    pltpu.sync_copy(vb, table_sh.at[rs])
    pltpu.sync_copy(vc, acc_sh.at[rs])
    pltpu.sync_copy(adj_hbm.at[0].at[pl.ds(base, GPW)],
                    srcv.at[pl.ds(0, GPW)])
    pltpu.sync_copy(adj_hbm.at[1].at[pl.ds(base, GPW)],
                    dstv.at[pl.ds(0, GPW)])
    _load_extra_idx(adj_hbm, srcv, dstv, wid, base)
    plsc.subcore_barrier()

    _msg_pipeline(srcv, dstv, rowsbuf, table_sh, acc_sh, gsem, ssem, wid)

    plsc.subcore_barrier()
    pltpu.sync_copy(acc_sh.at[rs], vd)

    @pl.loop(0, PKW)
    def _(i):
        for u in range(8):
            r = i * 8 + u
            vh[i, pl.ds(u * D_MSG, D_MSG)] = (vd[r] + 0.5 * vb[r]) * va[r]

    pltpu.sync_copy(vh, out_hbm.at[cid].at[pl.ds(sid * PKW, PKW)])


_deg_kernel = pl.kernel(
    _deg_body,
    out_type=jax.ShapeDtypeStruct((NC, NP, D_MSG), jnp.float32),
    mesh=_mesh,
    scratch_types=[
        pltpu.VMEM((GPW + 1, GSZ), jnp.int32),
        pltpu.VMEM((GPW + 1, GSZ), jnp.int32),
        pltpu.VMEM((GSZ, D_MSG), jnp.float32),
        pltpu.VMEM_SHARED((NP, D_MSG), jnp.float32),
        pltpu.SemaphoreType.DMA,
    ],
    compiler_params=_sc_params,
)

_msg1_kernel = pl.kernel(
    _msg1_body,
    out_type=jax.ShapeDtypeStruct((NC, NP, D_MSG), jnp.float32),
    mesh=_mesh,
    scratch_types=[
        pltpu.VMEM((GPW + 1, GSZ), jnp.int32),
        pltpu.VMEM((GPW + 1, GSZ), jnp.int32),
        pltpu.VMEM((2, CH, GSZ, D_MSG), jnp.float32),
        pltpu.VMEM((RPW, D_MSG), jnp.float32),
        pltpu.VMEM((RPW, D_MSG), jnp.float32),
        pltpu.VMEM((PKW, GSZ), jnp.float32),
        pltpu.VMEM_SHARED((NP, D_MSG), jnp.float32),
        pltpu.VMEM_SHARED((NP, D_MSG), jnp.float32),
        pltpu.SemaphoreType.DMA,
        pltpu.SemaphoreType.DMA,
    ],
    compiler_params=_sc_params,
)

_msg2_kernel = pl.kernel(
    _msg2_body,
    out_type=jax.ShapeDtypeStruct((NC, NP // 8, GSZ), jnp.float32),
    mesh=_mesh,
    scratch_types=[
        pltpu.VMEM((GPW + 1, GSZ), jnp.int32),
        pltpu.VMEM((GPW + 1, GSZ), jnp.int32),
        pltpu.VMEM((2, CH, GSZ, D_MSG), jnp.float32),
        pltpu.VMEM((RPW, D_MSG), jnp.float32),
        pltpu.VMEM((RPW, D_MSG), jnp.float32),
        pltpu.VMEM((RPW, D_MSG), jnp.float32),
        pltpu.VMEM((RPW, D_MSG), jnp.float32),
        pltpu.VMEM((PKW, GSZ), jnp.float32),
        pltpu.VMEM_SHARED((NP, D_MSG), jnp.float32),
        pltpu.VMEM_SHARED((NP, D_MSG), jnp.float32),
        pltpu.SemaphoreType.DMA,
        pltpu.SemaphoreType.DMA,
    ],
    compiler_params=_sc_params,
)


def _mm1_body(x_ref, w_ref, o_ref):
    o_ref[...] = jnp.dot(x_ref[...], w_ref[...],
                         preferred_element_type=jnp.float32)


def _fin_body(m_ref, w2_ref, o_ref):
    o_ref[...] = jnp.dot(m_ref[0] + m_ref[1], w2_ref[...],
                         preferred_element_type=jnp.float32)


def kernel(x, adj, W1, W2):
    n = x.shape[0]
    d_out = W2.shape[1]
    adj3d = adj.astype(jnp.int32).reshape(2, GROUPS, GSZ)
    zeros = jnp.zeros((NP, D_MSG), jnp.float32)
    ones = jnp.ones((GSZ, D_MSG), jnp.float32)

    degp = _deg_kernel(adj3d, ones, zeros)
    h1 = pl.pallas_call(
        _mm1_body,
        out_shape=jax.ShapeDtypeStruct((n, D_MSG), jnp.float32),
    )(x, W1)
    h1pk = jnp.concatenate(
        [h1, jnp.zeros((NP - n, D_MSG), jnp.float32)]).reshape(NP // 8, GSZ)

    acc1 = _msg1_kernel(degp, h1pk, adj3d)
    m = _msg2_kernel(degp, h1pk, acc1, adj3d)

    w2blk = jnp.kron(jnp.eye(8, dtype=jnp.float32), W2)
    out8 = pl.pallas_call(
        _fin_body,
        out_shape=jax.ShapeDtypeStruct((NP // 8, 8 * d_out), jnp.float32),
    )(m, w2blk)
    return out8.reshape(NP, d_out)[:n]


# speedup vs baseline: 1.0141x; 1.0141x over previous
import jax
import jax.numpy as jnp
from jax import lax
from jax.experimental import pallas as pl
from jax.experimental.pallas import tpu as pltpu
from jax.experimental.pallas import tpu_sc as plsc

N_NODES = 10000
N_EDGES = 320000
D_MSG = 16

NP = 10240
GSZ = 128
GROUPS = N_EDGES // GSZ
NC, NS = 2, 16
NW = NC * NS
GPW = 78
CH1 = 13
CH2 = 6
RPW = NP // NS
PKW = RPW // 8

_mesh = plsc.VectorSubcoreMesh(core_axis_name="c", subcore_axis_name="s")
_sc_params = pltpu.CompilerParams(use_tc_tiling_on_sc=False,
                                  needs_layout_passes=False)


def _newton_rsqrt(d):
    iv = plsc.bitcast(d, jnp.int32)
    y = plsc.bitcast(jnp.int32(0x5F3759DF) - (iv >> 1), jnp.float32)
    for _ in range(3):
        y = y * (1.5 - 0.5 * d * y * y)
    return y


def _worker_base(wid):
    return wid * GPW + jnp.minimum(wid, 4)


def _load_extra_idx(adj_hbm, srcv, dstv, wid, base):
    @pl.when(wid < 4)
    def _():
        pltpu.sync_copy(adj_hbm.at[0].at[pl.ds(base + GPW, 1)],
                        srcv.at[pl.ds(GPW, 1)])
        pltpu.sync_copy(adj_hbm.at[1].at[pl.ds(base + GPW, 1)],
                        dstv.at[pl.ds(GPW, 1)])


def _msg_pipeline(srcv, dstv, rowsbuf, table_sh, acc_sh, gsem, ssem, wid, ch):
    nchunk = GPW // ch

    @pl.when(wid < 4)
    def _():
        pltpu.async_copy(table_sh.at[srcv.at[GPW]],
                         rowsbuf.at[0].at[0], gsem).wait()
        pltpu.async_copy(rowsbuf.at[0].at[0], acc_sh.at[dstv.at[GPW]],
                         ssem, add=True).wait()

    def fire_gathers(c, b):
        return [pltpu.async_copy(table_sh.at[srcv.at[c * ch + j]],
                                 rowsbuf.at[b].at[j], gsem)
                for j in range(ch)]

    def fire_scatters(c, b):
        return [pltpu.async_copy(rowsbuf.at[b].at[j],
                                 acc_sh.at[dstv.at[c * ch + j]], ssem,
                                 add=True)
                for j in range(ch)]

    g = fire_gathers(0, 0)
    s_prev = None
    for c in range(nchunk):
        b = c % 2
        for d_ in g:
            d_.wait()
        s = fire_scatters(c, b)
        if s_prev is not None:
            for d_ in s_prev:
                d_.wait()
        if c + 1 < nchunk:
            g = fire_gathers(c + 1, 1 - b)
        s_prev = s
    for d_ in s_prev:
        d_.wait()


def _deg_body(adj_hbm, ones_hbm, zeros_hbm, out_hbm, dstv, srcv, onesb,
              acc_sh, ssem):
    cid = lax.axis_index("c")
    sid = lax.axis_index("s")
    wid = sid * NC + cid
    base = _worker_base(wid)
    rs = pl.ds(sid * RPW, RPW)

    pltpu.sync_copy(zeros_hbm.at[rs], acc_sh.at[rs])
    pltpu.sync_copy(ones_hbm, onesb)
    pltpu.sync_copy(adj_hbm.at[1].at[pl.ds(base, GPW)],
                    dstv.at[pl.ds(0, GPW)])
    _load_extra_idx(adj_hbm, srcv, dstv, wid, base)
    plsc.subcore_barrier()

    @pl.when(wid < 4)
    def _():
        pltpu.async_copy(onesb, acc_sh.at[dstv.at[GPW]], ssem,
                         add=True).wait()

    s_prev = None
    for c in range(GPW // CH1):
        s = [pltpu.async_copy(onesb, acc_sh.at[dstv.at[c * CH1 + j]], ssem,
                              add=True)
             for j in range(CH1)]
        if s_prev is not None:
            for d_ in s_prev:
                d_.wait()
        s_prev = s
    for d_ in s_prev:
        d_.wait()

    plsc.subcore_barrier()
    pltpu.sync_copy(acc_sh.at[rs], out_hbm.at[cid].at[rs])


def _msg1_body(degp_hbm, h1_hbm, adj_hbm, out_hbm,
               srcv, dstv, rowsbuf, va, vb, vh, acc_sh, table_sh,
               gsem, ssem):
    cid = lax.axis_index("c")
    sid = lax.axis_index("s")
    wid = sid * NC + cid
    base = _worker_base(wid)
    rs = pl.ds(sid * RPW, RPW)

    pltpu.sync_copy(degp_hbm.at[0].at[rs], va)
    pltpu.sync_copy(degp_hbm.at[1].at[rs], vb)
    pltpu.sync_copy(h1_hbm.at[pl.ds(sid * PKW, PKW)], vh)

    @pl.loop(0, PKW)
    def _(i):
        for u in range(8):
            r = i * 8 + u
            y = _newton_rsqrt(va[r] + vb[r] + 1.0)
            vb[r] = vh[i, pl.ds(u * D_MSG, D_MSG)] * y
            va[r] = jnp.zeros((D_MSG,), jnp.float32)

    pltpu.sync_copy(vb, table_sh.at[rs])
    pltpu.sync_copy(va, acc_sh.at[rs])
    pltpu.sync_copy(adj_hbm.at[0].at[pl.ds(base, GPW)],
                    srcv.at[pl.ds(0, GPW)])
    pltpu.sync_copy(adj_hbm.at[1].at[pl.ds(base, GPW)],
                    dstv.at[pl.ds(0, GPW)])
    _load_extra_idx(adj_hbm, srcv, dstv, wid, base)
    plsc.subcore_barrier()

    _msg_pipeline(srcv, dstv, rowsbuf, table_sh, acc_sh, gsem, ssem, wid, CH1)

    plsc.subcore_barrier()
    pltpu.sync_copy(acc_sh.at[rs], out_hbm.at[cid].at[rs])


def _msg2_body(degp_hbm, h1_hbm, acc1_hbm, adj_hbm, out_hbm,
               srcv, dstv, rowsbuf, va, vb, vc, vd, vh, acc_sh, table_sh,
               gsem, ssem):
    cid = lax.axis_index("c")
    sid = lax.axis_index("s")
    wid = sid * NC + cid
    base = _worker_base(wid)
    rs = pl.ds(sid * RPW, RPW)

    pltpu.sync_copy(degp_hbm.at[0].at[rs], va)
    pltpu.sync_copy(degp_hbm.at[1].at[rs], vb)
    pltpu.sync_copy(acc1_hbm.at[0].at[rs], vc)
    pltpu.sync_copy(acc1_hbm.at[1].at[rs], vd)
    pltpu.sync_copy(h1_hbm.at[pl.ds(sid * PKW, PKW)], vh)

    @pl.loop(0, PKW)
    def _(i):
        for u in range(8):
            r = i * 8 + u
            y = _newton_rsqrt(va[r] + vb[r] + 1.0)
            t = (vc[r] + vd[r]) * y + vh[i, pl.ds(u * D_MSG, D_MSG)] * (y * y)
            va[r] = y
            vb[r] = jnp.maximum(t, 0.0) * y
            vc[r] = jnp.zeros((D_MSG,), jnp.float32)

    pltpu.sync_copy(vb, table_sh.at[rs])
    pltpu.sync_copy(vc, acc_sh.at[rs])
    pltpu.sync_copy(adj_hbm.at[0].at[pl.ds(base, GPW)],
                    srcv.at[pl.ds(0, GPW)])
    pltpu.sync_copy(adj_hbm.at[1].at[pl.ds(base, GPW)],
                    dstv.at[pl.ds(0, GPW)])
    _load_extra_idx(adj_hbm, srcv, dstv, wid, base)
    plsc.subcore_barrier()

    _msg_pipeline(srcv, dstv, rowsbuf, table_sh, acc_sh, gsem, ssem, wid, CH2)

    plsc.subcore_barrier()
    pltpu.sync_copy(acc_sh.at[rs], vd)

    @pl.loop(0, PKW)
    def _(i):
        for u in range(8):
            r = i * 8 + u
            vh[i, pl.ds(u * D_MSG, D_MSG)] = (vd[r] + 0.5 * vb[r]) * va[r]

    pltpu.sync_copy(vh, out_hbm.at[cid].at[pl.ds(sid * PKW, PKW)])


_deg_kernel = pl.kernel(
    _deg_body,
    out_type=jax.ShapeDtypeStruct((NC, NP, D_MSG), jnp.float32),
    mesh=_mesh,
    scratch_types=[
        pltpu.VMEM((GPW + 1, GSZ), jnp.int32),
        pltpu.VMEM((GPW + 1, GSZ), jnp.int32),
        pltpu.VMEM((GSZ, D_MSG), jnp.float32),
        pltpu.VMEM_SHARED((NP, D_MSG), jnp.float32),
        pltpu.SemaphoreType.DMA,
    ],
    compiler_params=_sc_params,
)

_msg1_kernel = pl.kernel(
    _msg1_body,
    out_type=jax.ShapeDtypeStruct((NC, NP, D_MSG), jnp.float32),
    mesh=_mesh,
    scratch_types=[
        pltpu.VMEM((GPW + 1, GSZ), jnp.int32),
        pltpu.VMEM((GPW + 1, GSZ), jnp.int32),
        pltpu.VMEM((2, CH1, GSZ, D_MSG), jnp.float32),
        pltpu.VMEM((RPW, D_MSG), jnp.float32),
        pltpu.VMEM((RPW, D_MSG), jnp.float32),
        pltpu.VMEM((PKW, GSZ), jnp.float32),
        pltpu.VMEM_SHARED((NP, D_MSG), jnp.float32),
        pltpu.VMEM_SHARED((NP, D_MSG), jnp.float32),
        pltpu.SemaphoreType.DMA,
        pltpu.SemaphoreType.DMA,
    ],
    compiler_params=_sc_params,
)

_msg2_kernel = pl.kernel(
    _msg2_body,
    out_type=jax.ShapeDtypeStruct((NC, NP // 8, GSZ), jnp.float32),
    mesh=_mesh,
    scratch_types=[
        pltpu.VMEM((GPW + 1, GSZ), jnp.int32),
        pltpu.VMEM((GPW + 1, GSZ), jnp.int32),
        pltpu.VMEM((2, CH2, GSZ, D_MSG), jnp.float32),
        pltpu.VMEM((RPW, D_MSG), jnp.float32),
        pltpu.VMEM((RPW, D_MSG), jnp.float32),
        pltpu.VMEM((RPW, D_MSG), jnp.float32),
        pltpu.VMEM((RPW, D_MSG), jnp.float32),
        pltpu.VMEM((PKW, GSZ), jnp.float32),
        pltpu.VMEM_SHARED((NP, D_MSG), jnp.float32),
        pltpu.VMEM_SHARED((NP, D_MSG), jnp.float32),
        pltpu.SemaphoreType.DMA,
        pltpu.SemaphoreType.DMA,
    ],
    compiler_params=_sc_params,
)


def _mm1_body(x_ref, w_ref, o_ref):
    o_ref[...] = jnp.dot(x_ref[...], w_ref[...],
                         preferred_element_type=jnp.float32)


def _fin_body(m_ref, w2_ref, o_ref):
    o_ref[...] = jnp.dot(m_ref[0] + m_ref[1], w2_ref[...],
                         preferred_element_type=jnp.float32)


def kernel(x, adj, W1, W2):
    n = x.shape[0]
    d_out = W2.shape[1]
    adj3d = adj.astype(jnp.int32).reshape(2, GROUPS, GSZ)
    zeros = jnp.zeros((NP, D_MSG), jnp.float32)
    ones = jnp.ones((GSZ, D_MSG), jnp.float32)

    degp = _deg_kernel(adj3d, ones, zeros)
    h1 = pl.pallas_call(
        _mm1_body,
        out_shape=jax.ShapeDtypeStruct((n, D_MSG), jnp.float32),
    )(x, W1)
    h1pk = jnp.concatenate(
        [h1, jnp.zeros((NP - n, D_MSG), jnp.float32)]).reshape(NP // 8, GSZ)

    acc1 = _msg1_kernel(degp, h1pk, adj3d)
    m = _msg2_kernel(degp, h1pk, acc1, adj3d)

    w2blk = jnp.kron(jnp.eye(8, dtype=jnp.float32), W2)
    out8 = pl.pallas_call(
        _fin_body,
        out_shape=jax.ShapeDtypeStruct((NP // 8, 8 * d_out), jnp.float32),
    )(m, w2blk)
    return out8.reshape(NP, d_out)[:n]
